# Initial kernel scaffold; baseline (speedup 1.0000x reference)
import functools

import jax
import jax.numpy as jnp
from jax import lax
from jax.experimental import pallas as pl
from jax.experimental.pallas import tpu as pltpu
from jax.experimental.pallas import tpu_sc as plsc

NC = 2
NS = 16
NW = NC * NS

EMB_DIM = 64
IDX_PER_GATHER = 128


def _make_gather(num_rows_total: int):
    assert num_rows_total % (NW * IDX_PER_GATHER) == 0
    rows_per_w = num_rows_total // NW
    r = rows_per_w // IDX_PER_GATHER

    mesh = plsc.VectorSubcoreMesh(core_axis_name="c", subcore_axis_name="s")

    @functools.partial(
        pl.kernel,
        out_type=jax.ShapeDtypeStruct((num_rows_total, EMB_DIM), jnp.float32),
        mesh=mesh,
        scratch_types=[
            pltpu.VMEM((r, IDX_PER_GATHER), jnp.int32),
            pltpu.VMEM((IDX_PER_GATHER, EMB_DIM), jnp.float32),
            pltpu.SemaphoreType.DMA,
        ],
    )
    def gather_kernel(idx_hbm, table_hbm, out_hbm, idx_v, rows_v, sem):
        wid = lax.axis_index("s") * NC + lax.axis_index("c")
        pltpu.sync_copy(idx_hbm.at[pl.ds(wid * r, r)], idx_v)
        base = wid * rows_per_w

        def step(j, carry):
            pltpu.async_copy(table_hbm.at[idx_v.at[j]], rows_v, sem).wait()
            pltpu.sync_copy(
                rows_v,
                out_hbm.at[pl.ds(base + j * IDX_PER_GATHER, IDX_PER_GATHER)],
            )
            return carry

        lax.fori_loop(0, r, step, 0)

    return gather_kernel


def kernel(input, weight):
    b, s = input.shape
    n = b * s
    idx2d = input.reshape(n // IDX_PER_GATHER, IDX_PER_GATHER).astype(jnp.int32)
    out = _make_gather(n)(idx2d, weight)
    return out.reshape(b, s, EMB_DIM)


# speedup vs baseline: 4.0793x; 4.0793x over previous
import functools

import jax
import jax.numpy as jnp
from jax import lax
from jax.experimental import pallas as pl
from jax.experimental.pallas import tpu as pltpu
from jax.experimental.pallas import tpu_sc as plsc

NC = 2
NS = 16
NW = NC * NS

EMB_DIM = 64
IDX_PER_GATHER = 128


def _make_gather(num_rows_total: int):
    assert num_rows_total % (NW * IDX_PER_GATHER) == 0
    rows_per_w = num_rows_total // NW
    r = rows_per_w // IDX_PER_GATHER

    mesh = plsc.VectorSubcoreMesh(core_axis_name="c", subcore_axis_name="s")

    @functools.partial(
        pl.kernel,
        out_type=jax.ShapeDtypeStruct(
            (NW, r, IDX_PER_GATHER, EMB_DIM), jnp.float32
        ),
        mesh=mesh,
        scratch_types=[
            pltpu.VMEM((r, IDX_PER_GATHER), jnp.int32),
            pltpu.VMEM((IDX_PER_GATHER, EMB_DIM), jnp.float32),
            pltpu.SemaphoreType.DMA,
        ],
        compiler_params=pltpu.CompilerParams(use_tc_tiling_on_sc=False),
    )
    def gather_kernel(idx_hbm, table_hbm, out_hbm, idx_v, rows_v, sem):
        wid = lax.axis_index("s") * NC + lax.axis_index("c")
        pltpu.sync_copy(idx_hbm.at[wid], idx_v)

        def step(j, carry):
            pltpu.async_copy(table_hbm.at[idx_v.at[j]], rows_v, sem).wait()
            pltpu.sync_copy(rows_v, out_hbm.at[wid, j])
            return carry

        lax.fori_loop(0, r, step, 0)

    return gather_kernel


def kernel(input, weight):
    b, s = input.shape
    n = b * s
    r = n // (NW * IDX_PER_GATHER)
    idx3d = input.reshape(NW, r, IDX_PER_GATHER).astype(jnp.int32)
    out = _make_gather(n)(idx3d, weight)
    return out.reshape(b, s, EMB_DIM)


# speedup vs baseline: 4.6064x; 1.1292x over previous
import functools

import jax
import jax.numpy as jnp
from jax import lax
from jax.experimental import pallas as pl
from jax.experimental.pallas import tpu as pltpu
from jax.experimental.pallas import tpu_sc as plsc

NC = 2
NS = 16
NW = NC * NS

EMB_DIM = 64
IDX_PER_GATHER = 128
K = 5


def _make_gather(num_rows_total: int):
    assert num_rows_total % (NW * IDX_PER_GATHER) == 0
    rows_per_w = num_rows_total // NW
    r = rows_per_w // IDX_PER_GATHER
    assert r % (2 * K) == 0
    half_iters = r // (2 * K)

    mesh = plsc.VectorSubcoreMesh(core_axis_name="c", subcore_axis_name="s")

    @functools.partial(
        pl.kernel,
        out_type=jax.ShapeDtypeStruct((NW, r, IDX_PER_GATHER, EMB_DIM),
                                      jnp.float32),
        mesh=mesh,
        scratch_types=[
            pltpu.VMEM((r, IDX_PER_GATHER), jnp.int32),
            pltpu.VMEM((K, IDX_PER_GATHER, EMB_DIM), jnp.float32),
            pltpu.VMEM((K, IDX_PER_GATHER, EMB_DIM), jnp.float32),
            pltpu.SemaphoreType.DMA,
            pltpu.SemaphoreType.DMA,
            pltpu.SemaphoreType.DMA,
            pltpu.SemaphoreType.DMA,
        ],
        compiler_params=pltpu.CompilerParams(use_tc_tiling_on_sc=False),
    )
    def gather_kernel(idx_hbm, table_hbm, out_hbm, idx_v, buf0, buf1,
                      gs0, gs1, ws0, ws1):
        wid = lax.axis_index("s") * NC + lax.axis_index("c")
        pltpu.sync_copy(idx_hbm.at[wid], idx_v)

        def fire_group(g, buf, gsem):
            for k in range(K):
                pltpu.async_copy(table_hbm.at[idx_v.at[g * K + k]],
                                 buf.at[k], gsem)

        def drain_group(buf, gsem):
            for k in range(K):
                pltpu.make_async_copy(table_hbm.at[idx_v.at[k]],
                                      buf.at[k], gsem).wait()

        def wb(g, buf, wsem):
            return pltpu.async_copy(buf, out_hbm.at[wid, pl.ds(g * K, K)],
                                    wsem)

        def wb_wait(buf, wsem):
            pltpu.make_async_copy(buf, out_hbm.at[wid, pl.ds(0, K)],
                                  wsem).wait()

        def body(i, carry):
            g0 = 2 * i
            g1 = 2 * i + 1

            @pl.when(i > 0)
            def _():
                wb_wait(buf0, ws0)

            fire_group(g0, buf0, gs0)

            @pl.when(i > 0)
            def _():
                wb_wait(buf1, ws1)

            fire_group(g1, buf1, gs1)

            drain_group(buf0, gs0)
            wb(g0, buf0, ws0)
            drain_group(buf1, gs1)
            wb(g1, buf1, ws1)
            return carry

        lax.fori_loop(0, half_iters, body, 0)
        wb_wait(buf0, ws0)
        wb_wait(buf1, ws1)

    return gather_kernel


def kernel(input, weight):
    b, s = input.shape
    n = b * s
    r = n // (NW * IDX_PER_GATHER)
    idx3d = input.reshape(NW, r, IDX_PER_GATHER).astype(jnp.int32)
    out = _make_gather(n)(idx3d, weight)
    return out.reshape(b, s, EMB_DIM)
